# baseline (speedup 1.0000x reference)
import functools

import jax
import jax.numpy as jnp
from jax import lax
from jax.experimental import pallas as pl
from jax.experimental.pallas import tpu as pltpu
from jax.experimental.pallas import tpu_sc as plsc

B = 16
N = 2048
D = 3

NC = 2
NS = 16
NW = NC * NS

TOK = B * N
TOK_W = TOK // NW
ELEM_W = TOK_W * D
GROUPS = TOK_W // 16

_mesh = plsc.VectorSubcoreMesh(core_axis_name="c", subcore_axis_name="s")


@functools.partial(
    pl.kernel,
    out_type=(
        jax.ShapeDtypeStruct((TOK * D,), jnp.float32),
        jax.ShapeDtypeStruct((TOK,), jnp.int32),
    ),
    mesh=_mesh,
    scratch_types=[
        pltpu.VMEM((ELEM_W,), jnp.float32),
        pltpu.VMEM((ELEM_W,), jnp.float32),
        pltpu.VMEM((TOK_W,), jnp.int32),
    ],
    compiler_params=pltpu.CompilerParams(needs_layout_passes=False),
)
def _sc_embed(x_hbm, out_hbm, mask_hbm, xv, ov, mv):
    wid = lax.axis_index("s") * NC + lax.axis_index("c")
    ebase = wid * ELEM_W
    tbase = wid * TOK_W

    pltpu.sync_copy(x_hbm.at[pl.ds(ebase, ELEM_W)], xv)

    lane = lax.iota(jnp.int32, 16)
    conf_idx = [((16 * k + lane) // 3) * 3 + 2 for k in range(D)]
    tok_conf_idx = 3 * lane + 2
    zf = jnp.zeros((16,), jnp.float32)
    zi = jnp.zeros((16,), jnp.int32)
    oi = jnp.ones((16,), jnp.int32)

    def body(g, _):
        base = 48 * g
        c_tok = plsc.load_gather(xv, [base + tok_conf_idx])
        mv[pl.ds(16 * g, 16)] = jnp.where(c_tok > zf, oi, zi)
        for k in range(D):
            c = plsc.load_gather(xv, [base + conf_idx[k]])
            v = xv[pl.ds(base + 16 * k, 16)]
            ov[pl.ds(base + 16 * k, 16)] = jnp.where(c > zf, v, zf)
        return 0

    lax.fori_loop(0, GROUPS, body, 0)

    pltpu.sync_copy(ov, out_hbm.at[pl.ds(ebase, ELEM_W)])
    pltpu.sync_copy(mv, mask_hbm.at[pl.ds(tbase, TOK_W)])


def kernel(x):
    out_flat, mask_i32 = _sc_embed(x.reshape(-1))
    out = out_flat.reshape(B, N, D)
    mask = mask_i32.reshape(B, N).astype(jnp.bool_)
    return (out, mask)


# speedup vs baseline: 2.2863x; 2.2863x over previous
import functools

import jax
import jax.numpy as jnp
from jax import lax
from jax.experimental import pallas as pl
from jax.experimental.pallas import tpu as pltpu
from jax.experimental.pallas import tpu_sc as plsc

B, N, D = 16, 2048, 3
NC, NS = 2, 16
NW = NC * NS
TOK = B * N
TOK_W = TOK // NW
ELEM_W = TOK_W * D
HALVES = N // TOK_W
GROUPS = TOK_W // 16

_mesh = plsc.VectorSubcoreMesh(core_axis_name="c", subcore_axis_name="s")


@functools.partial(
    pl.kernel,
    out_type=(
        jax.ShapeDtypeStruct((D, B, N), jnp.float32),
        jax.ShapeDtypeStruct((B, N), jnp.int32),
    ),
    mesh=_mesh,
    scratch_types=[
        pltpu.VMEM((ELEM_W,), jnp.float32),
        pltpu.VMEM((D, TOK_W), jnp.float32),
        pltpu.VMEM((TOK_W,), jnp.int32),
    ],
    compiler_params=pltpu.CompilerParams(needs_layout_passes=False),
)
def _sc(x_hbm, out_hbm, mask_hbm, xv, pv, mv):
    wid = lax.axis_index("s") * NC + lax.axis_index("c")
    b = wid // HALVES
    nbase = (wid % HALVES) * TOK_W

    pltpu.sync_copy(x_hbm.at[b, pl.ds(nbase * D, ELEM_W)], xv)

    lane = lax.iota(jnp.int32, 16)
    tok3 = 3 * lane
    zf = jnp.zeros((16,), jnp.float32)
    zi = jnp.zeros((16,), jnp.int32)
    oi = jnp.ones((16,), jnp.int32)

    def body(g, _):
        base = 48 * g
        c = plsc.load_gather(xv, [base + tok3 + 2])
        keep = c > zf
        v0 = plsc.load_gather(xv, [base + tok3])
        v1 = plsc.load_gather(xv, [base + tok3 + 1])
        pv[0, pl.ds(16 * g, 16)] = jnp.where(keep, v0, zf)
        pv[1, pl.ds(16 * g, 16)] = jnp.where(keep, v1, zf)
        pv[2, pl.ds(16 * g, 16)] = jnp.where(keep, c, zf)
        mv[pl.ds(16 * g, 16)] = jnp.where(keep, oi, zi)
        return 0

    lax.fori_loop(0, GROUPS, body, 0)

    pltpu.sync_copy(pv, out_hbm.at[:, b, pl.ds(nbase, TOK_W)])
    pltpu.sync_copy(mv, mask_hbm.at[b, pl.ds(nbase, TOK_W)])


def kernel(x):
    planes, mask_i32 = _sc(x)
    out = planes.transpose(1, 2, 0)
    mask = mask_i32.astype(jnp.bool_)
    return (out, mask)


# speedup vs baseline: 2.2944x; 1.0036x over previous
import functools

import jax
import jax.numpy as jnp
from jax import lax
from jax.experimental import pallas as pl
from jax.experimental.pallas import tpu as pltpu
from jax.experimental.pallas import tpu_sc as plsc

B, N, D = 16, 2048, 3
NC, NS = 2, 16
NW = NC * NS
TOK = B * N
TOK_W = TOK // NW
ELEM_W = TOK_W * D
HALVES = N // TOK_W
TOK_H = TOK_W // 2
ELEM_H = TOK_H * D
GROUPS_H = TOK_H // 16

_mesh = plsc.VectorSubcoreMesh(core_axis_name="c", subcore_axis_name="s")


@functools.partial(
    pl.kernel,
    out_type=(
        jax.ShapeDtypeStruct((D, B, N), jnp.float32),
        jax.ShapeDtypeStruct((B, N), jnp.int32),
    ),
    mesh=_mesh,
    scratch_types=[
        pltpu.VMEM((ELEM_W,), jnp.float32),
        pltpu.VMEM((D, TOK_W), jnp.float32),
        pltpu.VMEM((TOK_W,), jnp.int32),
        pltpu.SemaphoreType.DMA,
        pltpu.SemaphoreType.DMA,
        pltpu.SemaphoreType.DMA,
    ],
    compiler_params=pltpu.CompilerParams(needs_layout_passes=False),
)
def _sc(x_hbm, out_hbm, mask_hbm, xv, pv, mv, si0, si1, so):
    wid = lax.axis_index("s") * NC + lax.axis_index("c")
    b = wid // HALVES
    nbase = (wid % HALVES) * TOK_W

    in0 = pltpu.async_copy(
        x_hbm.at[b, pl.ds(nbase * D, ELEM_H)], xv.at[pl.ds(0, ELEM_H)], si0
    )
    in1 = pltpu.async_copy(
        x_hbm.at[b, pl.ds(nbase * D + ELEM_H, ELEM_H)],
        xv.at[pl.ds(ELEM_H, ELEM_H)],
        si1,
    )

    lane = lax.iota(jnp.int32, 16)
    tok3 = 3 * lane
    zf = jnp.zeros((16,), jnp.float32)
    zi = jnp.zeros((16,), jnp.int32)
    oi = jnp.ones((16,), jnp.int32)

    def make_body(off):
        def body(g, _):
            base = off * D + 48 * g
            t = off + 16 * g
            c = plsc.load_gather(xv, [base + tok3 + 2])
            keep = c > zf
            v0 = plsc.load_gather(xv, [base + tok3])
            v1 = plsc.load_gather(xv, [base + tok3 + 1])
            pv[0, pl.ds(t, 16)] = jnp.where(keep, v0, zf)
            pv[1, pl.ds(t, 16)] = jnp.where(keep, v1, zf)
            pv[2, pl.ds(t, 16)] = jnp.where(keep, c, zf)
            mv[pl.ds(t, 16)] = jnp.where(keep, oi, zi)
            return 0

        return body

    in0.wait()
    lax.fori_loop(0, GROUPS_H, make_body(0), 0)
    out0 = pltpu.async_copy(
        pv.at[:, pl.ds(0, TOK_H)], out_hbm.at[:, b, pl.ds(nbase, TOK_H)], so
    )
    in1.wait()
    lax.fori_loop(0, GROUPS_H, make_body(TOK_H), 0)
    out1 = pltpu.async_copy(
        pv.at[:, pl.ds(TOK_H, TOK_H)],
        out_hbm.at[:, b, pl.ds(nbase + TOK_H, TOK_H)],
        so,
    )
    pltpu.sync_copy(mv, mask_hbm.at[b, pl.ds(nbase, TOK_W)])
    out0.wait()
    out1.wait()


def kernel(x):
    planes, mask_i32 = _sc(x)
    out = planes.transpose(1, 2, 0)
    mask = mask_i32.astype(jnp.bool_)
    return (out, mask)


# speedup vs baseline: 2.2983x; 1.0017x over previous
import functools

import jax
import jax.numpy as jnp
from jax import lax
from jax.experimental import pallas as pl
from jax.experimental.pallas import tpu as pltpu
from jax.experimental.pallas import tpu_sc as plsc

B, N, D = 16, 2048, 3
NC, NS = 2, 16
NW = NC * NS
TOK = B * N
TOK_W = TOK // NW
ELEM_W = TOK_W * D
HALVES = N // TOK_W
TOK_H = TOK_W // 2
ELEM_H = TOK_H * D
GROUPS_H = TOK_H // 16

_mesh = plsc.VectorSubcoreMesh(core_axis_name="c", subcore_axis_name="s")


@functools.partial(
    pl.kernel,
    out_type=(
        jax.ShapeDtypeStruct((D, B, N), jnp.float32),
        jax.ShapeDtypeStruct((B, N), jnp.int32),
    ),
    mesh=_mesh,
    scratch_types=[
        pltpu.VMEM((ELEM_W,), jnp.float32),
        pltpu.VMEM((D, TOK_W), jnp.float32),
        pltpu.VMEM((TOK_W,), jnp.int32),
        pltpu.SemaphoreType.DMA,
        pltpu.SemaphoreType.DMA,
        pltpu.SemaphoreType.DMA,
    ],
    compiler_params=pltpu.CompilerParams(needs_layout_passes=False),
)
def _sc(x_hbm, out_hbm, mask_hbm, xv, pv, mv, si0, si1, so):
    wid = lax.axis_index("s") * NC + lax.axis_index("c")
    b = wid // HALVES
    nbase = (wid % HALVES) * TOK_W

    in0 = pltpu.async_copy(
        x_hbm.at[b, pl.ds(nbase * D, ELEM_H)], xv.at[pl.ds(0, ELEM_H)], si0
    )
    in1 = pltpu.async_copy(
        x_hbm.at[b, pl.ds(nbase * D + ELEM_H, ELEM_H)],
        xv.at[pl.ds(ELEM_H, ELEM_H)],
        si1,
    )

    lane = lax.iota(jnp.int32, 16)
    tok3 = 3 * lane
    zf = jnp.zeros((16,), jnp.float32)
    zi = jnp.zeros((16,), jnp.int32)
    oi = jnp.ones((16,), jnp.int32)

    def half(off):
        @plsc.parallel_loop(0, GROUPS_H, unroll=4)
        def body(g):
            base = off * D + 48 * g
            t = off + 16 * g
            c = plsc.load_gather(xv, [base + tok3 + 2])
            keep = c > zf
            v0 = plsc.load_gather(xv, [base + tok3])
            v1 = plsc.load_gather(xv, [base + tok3 + 1])
            pv[0, pl.ds(t, 16)] = jnp.where(keep, v0, zf)
            pv[1, pl.ds(t, 16)] = jnp.where(keep, v1, zf)
            pv[2, pl.ds(t, 16)] = jnp.where(keep, c, zf)
            mv[pl.ds(t, 16)] = jnp.where(keep, oi, zi)

    in0.wait()
    half(0)
    out0 = pltpu.async_copy(
        pv.at[:, pl.ds(0, TOK_H)], out_hbm.at[:, b, pl.ds(nbase, TOK_H)], so
    )
    in1.wait()
    half(TOK_H)
    out1 = pltpu.async_copy(
        pv.at[:, pl.ds(TOK_H, TOK_H)],
        out_hbm.at[:, b, pl.ds(nbase + TOK_H, TOK_H)],
        so,
    )
    pltpu.sync_copy(mv, mask_hbm.at[b, pl.ds(nbase, TOK_W)])
    out0.wait()
    out1.wait()


def kernel(x):
    planes, mask_i32 = _sc(x)
    out = planes.transpose(1, 2, 0)
    mask = mask_i32.astype(jnp.bool_)
    return (out, mask)


# speedup vs baseline: 2.4332x; 1.0587x over previous
import functools

import jax
import jax.numpy as jnp
from jax import lax
from jax.experimental import pallas as pl
from jax.experimental.pallas import tpu as pltpu
from jax.experimental.pallas import tpu_sc as plsc

B, N, D = 16, 2048, 3
NC, NS = 1, 16
NW = NC * NS
TOK = B * N
TOK_W = TOK // NW
ELEM_W = TOK_W * D
HALVES = N // TOK_W
TOK_H = TOK_W // 2
ELEM_H = TOK_H * D
GROUPS_H = TOK_H // 16

_mesh = plsc.VectorSubcoreMesh(core_axis_name="c", subcore_axis_name="s", num_cores=1)


@functools.partial(
    pl.kernel,
    out_type=(
        jax.ShapeDtypeStruct((D, B, N), jnp.float32),
        jax.ShapeDtypeStruct((B, N), jnp.int32),
    ),
    mesh=_mesh,
    scratch_types=[
        pltpu.VMEM((ELEM_W,), jnp.float32),
        pltpu.VMEM((D, TOK_W), jnp.float32),
        pltpu.VMEM((TOK_W,), jnp.int32),
        pltpu.SemaphoreType.DMA,
        pltpu.SemaphoreType.DMA,
        pltpu.SemaphoreType.DMA,
    ],
    compiler_params=pltpu.CompilerParams(needs_layout_passes=False),
)
def _sc(x_hbm, out_hbm, mask_hbm, xv, pv, mv, si0, si1, so):
    wid = lax.axis_index("s") * NC + lax.axis_index("c")
    b = wid // HALVES
    nbase = (wid % HALVES) * TOK_W

    in0 = pltpu.async_copy(
        x_hbm.at[b, pl.ds(nbase * D, ELEM_H)], xv.at[pl.ds(0, ELEM_H)], si0
    )
    in1 = pltpu.async_copy(
        x_hbm.at[b, pl.ds(nbase * D + ELEM_H, ELEM_H)],
        xv.at[pl.ds(ELEM_H, ELEM_H)],
        si1,
    )

    lane = lax.iota(jnp.int32, 16)
    tok3 = 3 * lane
    zf = jnp.zeros((16,), jnp.float32)
    zi = jnp.zeros((16,), jnp.int32)
    oi = jnp.ones((16,), jnp.int32)

    def half(off):
        @plsc.parallel_loop(0, GROUPS_H, unroll=4)
        def body(g):
            base = off * D + 48 * g
            t = off + 16 * g
            c = plsc.load_gather(xv, [base + tok3 + 2])
            keep = c > zf
            v0 = plsc.load_gather(xv, [base + tok3])
            v1 = plsc.load_gather(xv, [base + tok3 + 1])
            pv[0, pl.ds(t, 16)] = jnp.where(keep, v0, zf)
            pv[1, pl.ds(t, 16)] = jnp.where(keep, v1, zf)
            pv[2, pl.ds(t, 16)] = jnp.where(keep, c, zf)
            mv[pl.ds(t, 16)] = jnp.where(keep, oi, zi)

    in0.wait()
    half(0)
    out0 = pltpu.async_copy(
        pv.at[:, pl.ds(0, TOK_H)], out_hbm.at[:, b, pl.ds(nbase, TOK_H)], so
    )
    in1.wait()
    half(TOK_H)
    out1 = pltpu.async_copy(
        pv.at[:, pl.ds(TOK_H, TOK_H)],
        out_hbm.at[:, b, pl.ds(nbase + TOK_H, TOK_H)],
        so,
    )
    pltpu.sync_copy(mv, mask_hbm.at[b, pl.ds(nbase, TOK_W)])
    out0.wait()
    out1.wait()


def kernel(x):
    planes, mask_i32 = _sc(x)
    out = planes.transpose(1, 2, 0)
    mask = mask_i32.astype(jnp.bool_)
    return (out, mask)
